# baseline (speedup 1.0000x reference)
import functools

import jax
import jax.numpy as jnp
from jax import lax
from jax.experimental import pallas as pl
from jax.experimental.pallas import tpu as pltpu
from jax.experimental.pallas import tpu_sc as plsc

_BATCH = 4096
_K = 64
_NC = 2
_NS = 16
_NW = _NC * _NS
_BPW = _BATCH // _NW
_L = 16
_GROUPS = _BPW // _L
_G_B = 3.5

_mesh = plsc.VectorSubcoreMesh(core_axis_name="c", subcore_axis_name="s")


@functools.partial(
    pl.kernel,
    mesh=_mesh,
    out_type=jax.ShapeDtypeStruct((_BATCH,), jnp.float32),
    compiler_params=pltpu.CompilerParams(
        needs_layout_passes=False, use_tc_tiling_on_sc=False
    ),
    scratch_types=[
        pltpu.VMEM((_BPW,), jnp.int32),
        pltpu.VMEM((_BPW,), jnp.int32),
        pltpu.VMEM((_BPW, _K), jnp.float32),
        pltpu.VMEM((_BPW, _K), jnp.float32),
        pltpu.VMEM((_BPW,), jnp.float32),
        pltpu.VMEM((_BPW,), jnp.float32),
        pltpu.VMEM((_BPW,), jnp.float32),
        pltpu.SemaphoreType.DMA,
    ],
)
def _mf_sc(uid_hbm, iid_hbm, user_m_hbm, item_m_hbm, user_b_hbm, item_b_hbm,
           out_hbm, uid_v, iid_v, urows, irows, ub_v, ib_v, out_v, sem):
    wid = lax.axis_index("s") * _NC + lax.axis_index("c")
    base = wid * _BPW
    pltpu.sync_copy(uid_hbm.at[pl.ds(base, _BPW)], uid_v)
    pltpu.sync_copy(iid_hbm.at[pl.ds(base, _BPW)], iid_v)
    c1 = pltpu.async_copy(user_m_hbm.at[uid_v], urows, sem)
    c2 = pltpu.async_copy(item_m_hbm.at[iid_v], irows, sem)
    c3 = pltpu.async_copy(user_b_hbm.at[uid_v], ub_v, sem)
    c4 = pltpu.async_copy(item_b_hbm.at[iid_v], ib_v, sem)
    c1.wait()
    c2.wait()
    c3.wait()
    c4.wait()
    lane = lax.iota(jnp.int32, _L)
    for g in range(_GROUPS):
        s = pl.ds(g * _L, _L)
        acc = ub_v[s] + ib_v[s] + jnp.float32(_G_B)
        for j in range(_L):
            p = g * _L + j
            t = urows[p, pl.ds(0, _L)] * irows[p, pl.ds(0, _L)]
            for c in range(1, _K // _L):
                t = t + urows[p, pl.ds(c * _L, _L)] * irows[p, pl.ds(c * _L, _L)]
            acc = jnp.where(lane == j, acc + jnp.sum(t), acc)
        out_v[s] = acc
    pltpu.sync_copy(out_v, out_hbm.at[pl.ds(base, _BPW)])


def kernel(x, user_m, item_m, user_b, item_b):
    uid = x[:, 0]
    iid = x[:, 1]
    return _mf_sc(uid, iid, user_m, item_m, user_b, item_b)
